# Initial kernel scaffold; baseline (speedup 1.0000x reference)
import jax
import jax.numpy as jnp
from jax.experimental import pallas as pl


def kernel(x, edge_index, edge_features, W1, b1, W2, b2, W3, b3, W_root, b_root, bn_gamma, bn_beta):
    raise NotImplementedError("write your pallas kernel here")


# speedup vs baseline: 1.1868x; 1.1868x over previous
import jax
import jax.numpy as jnp
from jax import lax
from jax.experimental import pallas as pl
from jax.experimental.pallas import tpu as pltpu
from jax.experimental.pallas import tpu_sc as plsc

N = 10000
E = 160000
D_IN = 32
D_OUT = 32
D_EDGE = 16
H1 = 32
H2 = 64

NC = 2
NS = 16
NW = NC * NS
EP = 163840
EPW = EP // NW
GB = 1024
NCH = EPW // GB
SCB = 128
SPJ = EPW // SCB
SPC = GB // SCB
NPAD = 10240
NPS = NPAD // NS

_SC_PARAMS = pltpu.CompilerParams(use_tc_tiling_on_sc=False)


def _mesh():
    return plsc.VectorSubcoreMesh(core_axis_name="c", subcore_axis_name="s")


def _gather_count_body(x_hbm, src_hbm, dst3_hbm, ones_hbm, zeros_hbm,
                       xs_out, cnt_out, idx_v, rows_v, didx_v, ones_v, acc_c,
                       sem):
    c = lax.axis_index("c")
    s = lax.axis_index("s")
    wid = s * NC + c
    base = wid * EPW

    pltpu.sync_copy(zeros_hbm, acc_c.at[pl.ds(s * NPS, NPS)])
    pltpu.sync_copy(ones_hbm, ones_v)
    pltpu.sync_copy(dst3_hbm.at[wid], didx_v)

    def chunk(j, carry):
        pltpu.sync_copy(src_hbm.at[pl.ds(base + j * GB, GB)], idx_v)

        def gjob(t, carry2):
            pltpu.async_copy(x_hbm.at[idx_v.at[pl.ds(t * SCB, SCB)]],
                             rows_v.at[pl.ds(t * SCB, SCB)], sem).wait()
            return carry2

        lax.fori_loop(0, SPC, gjob, 0)
        pltpu.sync_copy(rows_v, xs_out.at[pl.ds(base + j * GB, GB)])
        return carry

    lax.fori_loop(0, NCH, chunk, 0)

    plsc.subcore_barrier()

    def cjob(j, carry):
        pltpu.sync_copy(ones_v, acc_c.at[didx_v.at[j]], add=True)
        return carry

    lax.fori_loop(0, SPJ, cjob, 0)

    plsc.subcore_barrier()
    pltpu.sync_copy(acc_c.at[pl.ds(s * NPS, NPS)],
                    cnt_out.at[c, pl.ds(s * NPS, NPS)])


def _gather_count(x, src, dst3, ones_arr, zeros_arr):
    k = pl.kernel(
        _gather_count_body,
        out_type=(jax.ShapeDtypeStruct((EP, D_IN), jnp.float32),
                  jax.ShapeDtypeStruct((NC, NPAD, 16), jnp.float32)),
        mesh=_mesh(),
        scratch_types=[
            pltpu.VMEM((GB,), jnp.int32),
            pltpu.VMEM((GB, D_IN), jnp.float32),
            pltpu.VMEM((SPJ, SCB), jnp.int32),
            pltpu.VMEM((SCB, 16), jnp.float32),
            pltpu.VMEM_SHARED((NPAD, 16), jnp.float32),
            pltpu.SemaphoreType.DMA,
        ],
        compiler_params=_SC_PARAMS,
    )
    return k(x, src, dst3, ones_arr, zeros_arr)


def _scatter_body(msg_hbm, dst3_hbm, zeros_hbm, sum_out, msg_v, didx_v, acc_m):
    c = lax.axis_index("c")
    s = lax.axis_index("s")
    wid = s * NC + c
    base = wid * EPW

    pltpu.sync_copy(zeros_hbm, acc_m.at[pl.ds(s * NPS, NPS)])
    pltpu.sync_copy(dst3_hbm.at[wid], didx_v)
    plsc.subcore_barrier()

    def chunk(j, carry):
        pltpu.sync_copy(msg_hbm.at[pl.ds(base + j * GB, GB)], msg_v)

        def sjob(t, carry2):
            pltpu.sync_copy(msg_v.at[pl.ds(t * SCB, SCB)],
                            acc_m.at[didx_v.at[j * SPC + t]], add=True)
            return carry2

        lax.fori_loop(0, SPC, sjob, 0)
        return carry

    lax.fori_loop(0, NCH, chunk, 0)

    plsc.subcore_barrier()
    pltpu.sync_copy(acc_m.at[pl.ds(s * NPS, NPS)],
                    sum_out.at[c, pl.ds(s * NPS, NPS)])


def _scatter_sum(msg, dst3, zeros_arr):
    k = pl.kernel(
        _scatter_body,
        out_type=jax.ShapeDtypeStruct((NC, NPAD, D_OUT), jnp.float32),
        mesh=_mesh(),
        scratch_types=[
            pltpu.VMEM((GB, D_OUT), jnp.float32),
            pltpu.VMEM((SPJ, SCB), jnp.int32),
            pltpu.VMEM_SHARED((NPAD, D_OUT), jnp.float32),
        ],
        compiler_params=_SC_PARAMS,
    )
    return k(msg, dst3, zeros_arr)


BE = 2000


def _sigmoid(z):
    return 1.0 / (1.0 + jnp.exp(-z))


def _msg_body(ef_ref, xs_ref, w1_ref, b1_ref, w2_ref, b2_ref, w3_ref, b3_ref,
              msg_ref):
    ef = ef_ref[...]
    xs = xs_ref[...]
    h = _sigmoid(jnp.dot(ef, w1_ref[...],
                         preferred_element_type=jnp.float32) + b1_ref[...])
    h = _sigmoid(jnp.dot(h, w2_ref[...],
                         preferred_element_type=jnp.float32) + b2_ref[...])
    w = jnp.dot(h, w3_ref[...],
                preferred_element_type=jnp.float32) + b3_ref[...]
    w3d = w.reshape(BE, D_IN, D_OUT)
    msg_ref[...] = jnp.sum(w3d * xs[:, :, None], axis=1)


def _messages(ef, xs, W1, b1, W2, b2, W3, b3):
    grid = (E // BE,)
    full = lambda i: (0, 0)
    return pl.pallas_call(
        _msg_body,
        grid=grid,
        in_specs=[
            pl.BlockSpec((BE, D_EDGE), lambda i: (i, 0)),
            pl.BlockSpec((BE, D_IN), lambda i: (i, 0)),
            pl.BlockSpec((D_EDGE, H1), full),
            pl.BlockSpec((1, H1), full),
            pl.BlockSpec((H1, H2), full),
            pl.BlockSpec((1, H2), full),
            pl.BlockSpec((H2, D_IN * D_OUT), full),
            pl.BlockSpec((1, D_IN * D_OUT), full),
        ],
        out_specs=pl.BlockSpec((BE, D_OUT), lambda i: (i, 0)),
        out_shape=jax.ShapeDtypeStruct((EP, D_OUT), jnp.float32),
    )(ef, xs, W1, b1.reshape(1, H1), W2, b2.reshape(1, H2), W3,
      b3.reshape(1, D_IN * D_OUT))


def _final_body(sum_ref, cnt_ref, x_ref, wr_ref, br_ref, g_ref, bta_ref,
                out_ref):
    ssum = sum_ref[0] + sum_ref[1]
    cnt = cnt_ref[0, :, 0:1] + cnt_ref[1, :, 0:1]
    aggr = ssum / jnp.maximum(cnt, 1.0)
    out = aggr + jnp.dot(x_ref[...], wr_ref[...],
                         preferred_element_type=jnp.float32) + br_ref[...]
    out = jnp.where(out >= 0, out, 0.01 * out)
    mean = jnp.mean(out, axis=0, keepdims=True)
    cen = out - mean
    var = jnp.mean(cen * cen, axis=0, keepdims=True)
    out_ref[...] = g_ref[...] * cen * lax.rsqrt(var + 1e-5) + bta_ref[...]


def _finalize(sum_parts, cnt_parts, x, W_root, b_root, bn_gamma, bn_beta):
    full = lambda i: (0, 0)
    return pl.pallas_call(
        _final_body,
        grid=(1,),
        in_specs=[
            pl.BlockSpec((NC, N, D_OUT), lambda i: (0, 0, 0)),
            pl.BlockSpec((NC, N, 16), lambda i: (0, 0, 0)),
            pl.BlockSpec((N, D_IN), full),
            pl.BlockSpec((D_IN, D_OUT), full),
            pl.BlockSpec((1, D_OUT), full),
            pl.BlockSpec((1, D_OUT), full),
            pl.BlockSpec((1, D_OUT), full),
        ],
        out_specs=pl.BlockSpec((N, D_OUT), full),
        out_shape=jax.ShapeDtypeStruct((N, D_OUT), jnp.float32),
    )(sum_parts, cnt_parts, x, W_root, b_root.reshape(1, D_OUT),
      bn_gamma.reshape(1, D_OUT), bn_beta.reshape(1, D_OUT))


def kernel(x, edge_index, edge_features, W1, b1, W2, b2, W3, b3,
           W_root, b_root, bn_gamma, bn_beta):
    src = edge_index[0]
    dst = edge_index[1]
    src_p = jnp.concatenate([src, jnp.zeros((EP - E,), jnp.int32)])
    dst_p = jnp.concatenate([dst, jnp.full((EP - E,), N, jnp.int32)])
    dst3 = dst_p.reshape(NW, SPJ, SCB)

    ones_arr = jnp.zeros((SCB, 16), jnp.float32).at[:, 0].set(1.0)
    zeros16 = jnp.zeros((NPS, 16), jnp.float32)
    zeros32 = jnp.zeros((NPS, D_OUT), jnp.float32)

    xs, cnt_parts = _gather_count(x, src_p, dst3, ones_arr, zeros16)
    msg = _messages(edge_features, xs, W1, b1, W2, b2, W3, b3)
    sum_parts = _scatter_sum(msg, dst3, zeros32)
    out = _finalize(sum_parts, cnt_parts, x, W_root, b_root, bn_gamma, bn_beta)
    return (out, edge_index, edge_features)


# speedup vs baseline: 3.3571x; 2.8286x over previous
import jax
import jax.numpy as jnp
from jax import lax
from jax.experimental import pallas as pl
from jax.experimental.pallas import tpu as pltpu
from jax.experimental.pallas import tpu_sc as plsc

N = 10000
E = 160000
D_IN = 32
D_OUT = 32
D_EDGE = 16
H1 = 32
H2 = 64

NC = 2
NS = 16
NW = NC * NS
EP = 163840
EPW = EP // NW
GB = 1024
NCH = EPW // GB
SCB = 128
SPJ = EPW // SCB
SPC = GB // SCB
NPAD = 10240
NPS = NPAD // NS

_SC_PARAMS = pltpu.CompilerParams(use_tc_tiling_on_sc=False)


def _mesh():
    return plsc.VectorSubcoreMesh(core_axis_name="c", subcore_axis_name="s")


def _gather_count_body(x_hbm, src_hbm, dst3_hbm, ones_hbm, zeros_hbm,
                       xs_out, cnt_out, idx_v, rows_v, didx_v, ones_v, acc_c,
                       sem):
    c = lax.axis_index("c")
    s = lax.axis_index("s")
    wid = s * NC + c
    base = wid * EPW

    pltpu.sync_copy(zeros_hbm, acc_c.at[pl.ds(s * NPS, NPS)])
    pltpu.sync_copy(ones_hbm, ones_v)
    pltpu.sync_copy(dst3_hbm.at[wid], didx_v)

    def chunk(j, carry):
        pltpu.sync_copy(src_hbm.at[pl.ds(base + j * GB, GB)], idx_v)

        def gjob(t, carry2):
            pltpu.async_copy(x_hbm.at[idx_v.at[pl.ds(t * SCB, SCB)]],
                             rows_v.at[pl.ds(t * SCB, SCB)], sem).wait()
            return carry2

        lax.fori_loop(0, SPC, gjob, 0)
        pltpu.sync_copy(rows_v, xs_out.at[pl.ds(base + j * GB, GB)])
        return carry

    lax.fori_loop(0, NCH, chunk, 0)

    plsc.subcore_barrier()

    def cjob(j, carry):
        pltpu.sync_copy(ones_v, acc_c.at[didx_v.at[j]], add=True)
        return carry

    lax.fori_loop(0, SPJ, cjob, 0)

    plsc.subcore_barrier()
    pltpu.sync_copy(acc_c.at[pl.ds(s * NPS, NPS)],
                    cnt_out.at[c, pl.ds(s * NPS, NPS)])


def _gather_count(x, src, dst3, ones_arr, zeros_arr):
    k = pl.kernel(
        _gather_count_body,
        out_type=(jax.ShapeDtypeStruct((EP, D_IN), jnp.float32),
                  jax.ShapeDtypeStruct((NC, NPAD, 16), jnp.float32)),
        mesh=_mesh(),
        scratch_types=[
            pltpu.VMEM((GB,), jnp.int32),
            pltpu.VMEM((GB, D_IN), jnp.float32),
            pltpu.VMEM((SPJ, SCB), jnp.int32),
            pltpu.VMEM((SCB, 16), jnp.float32),
            pltpu.VMEM_SHARED((NPAD, 16), jnp.float32),
            pltpu.SemaphoreType.DMA,
        ],
        compiler_params=_SC_PARAMS,
    )
    return k(x, src, dst3, ones_arr, zeros_arr)


def _scatter_body(msg_hbm, dst3_hbm, zeros_hbm, sum_out, msg_v, didx_v, acc_m):
    c = lax.axis_index("c")
    s = lax.axis_index("s")
    wid = s * NC + c
    base = wid * EPW

    pltpu.sync_copy(zeros_hbm, acc_m.at[pl.ds(s * NPS, NPS)])
    pltpu.sync_copy(dst3_hbm.at[wid], didx_v)
    plsc.subcore_barrier()

    def chunk(j, carry):
        pltpu.sync_copy(msg_hbm.at[pl.ds(base + j * GB, GB)], msg_v)

        def sjob(t, carry2):
            pltpu.sync_copy(msg_v.at[pl.ds(t * SCB, SCB)],
                            acc_m.at[didx_v.at[j * SPC + t]], add=True)
            return carry2

        lax.fori_loop(0, SPC, sjob, 0)
        return carry

    lax.fori_loop(0, NCH, chunk, 0)

    plsc.subcore_barrier()
    pltpu.sync_copy(acc_m.at[pl.ds(s * NPS, NPS)],
                    sum_out.at[c, pl.ds(s * NPS, NPS)])


def _scatter_sum(msg, dst3, zeros_arr):
    k = pl.kernel(
        _scatter_body,
        out_type=jax.ShapeDtypeStruct((NC, NPAD, D_OUT), jnp.float32),
        mesh=_mesh(),
        scratch_types=[
            pltpu.VMEM((GB, D_OUT), jnp.float32),
            pltpu.VMEM((SPJ, SCB), jnp.int32),
            pltpu.VMEM_SHARED((NPAD, D_OUT), jnp.float32),
        ],
        compiler_params=_SC_PARAMS,
    )
    return k(msg, dst3, zeros_arr)


BE = 2000


def _sigmoid(z):
    return 1.0 / (1.0 + jnp.exp(-z))


def _msg_body(ef_ref, xs_ref, w1_ref, b1_ref, w2_ref, b2_ref, w3q_ref,
              b3q_ref, r_ref, msg_ref):
    ef = ef_ref[...]
    xs = xs_ref[...]
    fast = jax.lax.Precision.DEFAULT
    h = _sigmoid(jnp.dot(ef, w1_ref[...], precision=fast,
                         preferred_element_type=jnp.float32) + b1_ref[...])
    h = _sigmoid(jnp.dot(h, w2_ref[...], precision=fast,
                         preferred_element_type=jnp.float32) + b2_ref[...])
    w = jnp.dot(h, w3q_ref[...], precision=fast,
                preferred_element_type=jnp.float32) + b3q_ref[...]
    xt = pltpu.repeat(xs, D_OUT, axis=1)
    msg_ref[...] = jnp.dot(w * xt, r_ref[...], precision=fast,
                           preferred_element_type=jnp.float32)


def _messages(ef, xs, W1, b1, W2, b2, W3, b3):
    W3q = W3.reshape(H2, D_IN, D_OUT).transpose(0, 2, 1).reshape(H2, D_IN * D_OUT)
    b3q = b3.reshape(D_IN, D_OUT).T.reshape(1, D_IN * D_OUT)
    rmat = (jnp.arange(D_IN * D_OUT)[:, None] // D_IN
            == jnp.arange(D_OUT)[None, :]).astype(jnp.float32)
    grid = (E // BE,)
    full = lambda i: (0, 0)
    return pl.pallas_call(
        _msg_body,
        grid=grid,
        in_specs=[
            pl.BlockSpec((BE, D_EDGE), lambda i: (i, 0)),
            pl.BlockSpec((BE, D_IN), lambda i: (i, 0)),
            pl.BlockSpec((D_EDGE, H1), full),
            pl.BlockSpec((1, H1), full),
            pl.BlockSpec((H1, H2), full),
            pl.BlockSpec((1, H2), full),
            pl.BlockSpec((H2, D_IN * D_OUT), full),
            pl.BlockSpec((1, D_IN * D_OUT), full),
            pl.BlockSpec((D_IN * D_OUT, D_OUT), full),
        ],
        out_specs=pl.BlockSpec((BE, D_OUT), lambda i: (i, 0)),
        out_shape=jax.ShapeDtypeStruct((EP, D_OUT), jnp.float32),
    )(ef, xs, W1, b1.reshape(1, H1), W2, b2.reshape(1, H2), W3q, b3q, rmat)


def _final_body(sum_ref, cnt_ref, x_ref, wr_ref, br_ref, g_ref, bta_ref,
                out_ref):
    ssum = sum_ref[0] + sum_ref[1]
    cnt = cnt_ref[0, :, 0:1] + cnt_ref[1, :, 0:1]
    aggr = ssum / jnp.maximum(cnt, 1.0)
    out = aggr + jnp.dot(x_ref[...], wr_ref[...],
                         preferred_element_type=jnp.float32) + br_ref[...]
    out = jnp.where(out >= 0, out, 0.01 * out)
    mean = jnp.mean(out, axis=0, keepdims=True)
    cen = out - mean
    var = jnp.mean(cen * cen, axis=0, keepdims=True)
    out_ref[...] = g_ref[...] * cen * lax.rsqrt(var + 1e-5) + bta_ref[...]


def _finalize(sum_parts, cnt_parts, x, W_root, b_root, bn_gamma, bn_beta):
    full = lambda i: (0, 0)
    return pl.pallas_call(
        _final_body,
        grid=(1,),
        in_specs=[
            pl.BlockSpec((NC, N, D_OUT), lambda i: (0, 0, 0)),
            pl.BlockSpec((NC, N, 16), lambda i: (0, 0, 0)),
            pl.BlockSpec((N, D_IN), full),
            pl.BlockSpec((D_IN, D_OUT), full),
            pl.BlockSpec((1, D_OUT), full),
            pl.BlockSpec((1, D_OUT), full),
            pl.BlockSpec((1, D_OUT), full),
        ],
        out_specs=pl.BlockSpec((N, D_OUT), full),
        out_shape=jax.ShapeDtypeStruct((N, D_OUT), jnp.float32),
    )(sum_parts, cnt_parts, x, W_root, b_root.reshape(1, D_OUT),
      bn_gamma.reshape(1, D_OUT), bn_beta.reshape(1, D_OUT))


def kernel(x, edge_index, edge_features, W1, b1, W2, b2, W3, b3,
           W_root, b_root, bn_gamma, bn_beta):
    src = edge_index[0]
    dst = edge_index[1]
    src_p = jnp.concatenate([src, jnp.zeros((EP - E,), jnp.int32)])
    dst_p = jnp.concatenate([dst, jnp.full((EP - E,), N, jnp.int32)])
    dst3 = dst_p.reshape(NW, SPJ, SCB)

    ones_arr = jnp.zeros((SCB, 16), jnp.float32).at[:, 0].set(1.0)
    zeros16 = jnp.zeros((NPS, 16), jnp.float32)
    zeros32 = jnp.zeros((NPS, D_OUT), jnp.float32)

    xs, cnt_parts = _gather_count(x, src_p, dst3, ones_arr, zeros16)
    msg = _messages(edge_features, xs, W1, b1, W2, b2, W3, b3)
    sum_parts = _scatter_sum(msg, dst3, zeros32)
    out = _finalize(sum_parts, cnt_parts, x, W_root, b_root, bn_gamma, bn_beta)
    return (out, edge_index, edge_features)


# speedup vs baseline: 3.4920x; 1.0402x over previous
import jax
import jax.numpy as jnp
from jax import lax
from jax.experimental import pallas as pl
from jax.experimental.pallas import tpu as pltpu
from jax.experimental.pallas import tpu_sc as plsc

N = 10000
E = 160000
D_IN = 32
D_OUT = 32
D_EDGE = 16
H1 = 32
H2 = 64

NC = 2
NS = 16
NW = NC * NS
EP = 163840
EPW = EP // NW
GB = 1024
NCH = EPW // GB
SCB = 128
SPJ = EPW // SCB
SPC = GB // SCB
NPAD = 10240
NPS = NPAD // NS

_SC_PARAMS = pltpu.CompilerParams(use_tc_tiling_on_sc=False)


def _mesh():
    return plsc.VectorSubcoreMesh(core_axis_name="c", subcore_axis_name="s")


def _gather_count_body(x_hbm, src_hbm, dst3_hbm, ones_hbm, zeros_hbm,
                       xs_out, cnt_out, sidx_v, rows_a, rows_b, didx_v,
                       ones_v, acc_c, gsem_a, gsem_b, csem):
    c = lax.axis_index("c")
    s = lax.axis_index("s")
    wid = s * NC + c
    base = wid * EPW

    pltpu.sync_copy(zeros_hbm, acc_c.at[pl.ds(s * NPS, NPS)])
    pltpu.sync_copy(ones_hbm, ones_v)
    pltpu.sync_copy(dst3_hbm.at[wid], didx_v)
    pltpu.sync_copy(src_hbm.at[pl.ds(base, EPW)], sidx_v)

    bufs = (rows_a, rows_b)
    sems = (gsem_a, gsem_b)

    def fire(j, buf, sem):
        return [pltpu.async_copy(
                    x_hbm.at[sidx_v.at[pl.ds(j * GB + t * SCB, SCB)]],
                    buf.at[pl.ds(t * SCB, SCB)], sem)
                for t in range(SPC)]

    cps = fire(0, bufs[0], sems[0])
    for j in range(NCH):
        nxt = fire(j + 1, bufs[(j + 1) % 2], sems[(j + 1) % 2])             if j + 1 < NCH else None
        for cp in cps:
            cp.wait()
        pltpu.sync_copy(bufs[j % 2], xs_out.at[pl.ds(base + j * GB, GB)])
        cps = nxt

    plsc.subcore_barrier()

    for g in range(NCH):
        adds = [pltpu.async_copy(ones_v, acc_c.at[didx_v.at[g * SPC + t]],
                                 csem, add=True)
                for t in range(SPC)]
        for cp in adds:
            cp.wait()

    plsc.subcore_barrier()
    pltpu.sync_copy(acc_c.at[pl.ds(s * NPS, NPS)],
                    cnt_out.at[c, pl.ds(s * NPS, NPS)])


def _gather_count(x, src, dst3, ones_arr, zeros_arr):
    k = pl.kernel(
        _gather_count_body,
        out_type=(jax.ShapeDtypeStruct((EP, D_IN), jnp.float32),
                  jax.ShapeDtypeStruct((NC, NPAD, 16), jnp.float32)),
        mesh=_mesh(),
        scratch_types=[
            pltpu.VMEM((EPW,), jnp.int32),
            pltpu.VMEM((GB, D_IN), jnp.float32),
            pltpu.VMEM((GB, D_IN), jnp.float32),
            pltpu.VMEM((SPJ, SCB), jnp.int32),
            pltpu.VMEM((SCB, 16), jnp.float32),
            pltpu.VMEM_SHARED((NPAD, 16), jnp.float32),
            pltpu.SemaphoreType.DMA,
            pltpu.SemaphoreType.DMA,
            pltpu.SemaphoreType.DMA,
        ],
        compiler_params=_SC_PARAMS,
    )
    return k(x, src, dst3, ones_arr, zeros_arr)


def _scatter_body(msg_hbm, dst3_hbm, zeros_hbm, sum_out, msg_a, msg_b,
                  didx_v, acc_m, lsem_a, lsem_b, ssem):
    c = lax.axis_index("c")
    s = lax.axis_index("s")
    wid = s * NC + c
    base = wid * EPW

    pltpu.sync_copy(zeros_hbm, acc_m.at[pl.ds(s * NPS, NPS)])
    pltpu.sync_copy(dst3_hbm.at[wid], didx_v)
    plsc.subcore_barrier()

    bufs = (msg_a, msg_b)
    lsems = (lsem_a, lsem_b)
    lcp = pltpu.async_copy(msg_hbm.at[pl.ds(base, GB)], bufs[0], lsems[0])
    for j in range(NCH):
        nxt = pltpu.async_copy(msg_hbm.at[pl.ds(base + (j + 1) * GB, GB)],
                               bufs[(j + 1) % 2], lsems[(j + 1) % 2])             if j + 1 < NCH else None
        lcp.wait()
        adds = [pltpu.async_copy(bufs[j % 2].at[pl.ds(t * SCB, SCB)],
                                 acc_m.at[didx_v.at[j * SPC + t]], ssem,
                                 add=True)
                for t in range(SPC)]
        for cp in adds:
            cp.wait()
        lcp = nxt

    plsc.subcore_barrier()
    pltpu.sync_copy(acc_m.at[pl.ds(s * NPS, NPS)],
                    sum_out.at[c, pl.ds(s * NPS, NPS)])


def _scatter_sum(msg, dst3, zeros_arr):
    k = pl.kernel(
        _scatter_body,
        out_type=jax.ShapeDtypeStruct((NC, NPAD, D_OUT), jnp.float32),
        mesh=_mesh(),
        scratch_types=[
            pltpu.VMEM((GB, D_OUT), jnp.float32),
            pltpu.VMEM((GB, D_OUT), jnp.float32),
            pltpu.VMEM((SPJ, SCB), jnp.int32),
            pltpu.VMEM_SHARED((NPAD, D_OUT), jnp.float32),
            pltpu.SemaphoreType.DMA,
            pltpu.SemaphoreType.DMA,
            pltpu.SemaphoreType.DMA,
        ],
        compiler_params=_SC_PARAMS,
    )
    return k(msg, dst3, zeros_arr)


BE = 2000


def _sigmoid(z):
    return 1.0 / (1.0 + jnp.exp(-z))


def _msg_body(ef_ref, xs_ref, w1_ref, b1_ref, w2_ref, b2_ref, w3q_ref,
              b3q_ref, r_ref, msg_ref):
    ef = ef_ref[...]
    xs = xs_ref[...]
    fast = jax.lax.Precision.DEFAULT
    h = _sigmoid(jnp.dot(ef, w1_ref[...], precision=fast,
                         preferred_element_type=jnp.float32) + b1_ref[...])
    h = _sigmoid(jnp.dot(h, w2_ref[...], precision=fast,
                         preferred_element_type=jnp.float32) + b2_ref[...])
    w = jnp.dot(h, w3q_ref[...], precision=fast,
                preferred_element_type=jnp.float32) + b3q_ref[...]
    xt = pltpu.repeat(xs, D_OUT, axis=1)
    msg_ref[...] = jnp.dot(w * xt, r_ref[...], precision=fast,
                           preferred_element_type=jnp.float32)


def _messages(ef, xs, W1, b1, W2, b2, W3, b3):
    W3q = W3.reshape(H2, D_IN, D_OUT).transpose(0, 2, 1).reshape(H2, D_IN * D_OUT)
    b3q = b3.reshape(D_IN, D_OUT).T.reshape(1, D_IN * D_OUT)
    rmat = (jnp.arange(D_IN * D_OUT)[:, None] // D_IN
            == jnp.arange(D_OUT)[None, :]).astype(jnp.float32)
    grid = (E // BE,)
    full = lambda i: (0, 0)
    return pl.pallas_call(
        _msg_body,
        grid=grid,
        in_specs=[
            pl.BlockSpec((BE, D_EDGE), lambda i: (i, 0)),
            pl.BlockSpec((BE, D_IN), lambda i: (i, 0)),
            pl.BlockSpec((D_EDGE, H1), full),
            pl.BlockSpec((1, H1), full),
            pl.BlockSpec((H1, H2), full),
            pl.BlockSpec((1, H2), full),
            pl.BlockSpec((H2, D_IN * D_OUT), full),
            pl.BlockSpec((1, D_IN * D_OUT), full),
            pl.BlockSpec((D_IN * D_OUT, D_OUT), full),
        ],
        out_specs=pl.BlockSpec((BE, D_OUT), lambda i: (i, 0)),
        out_shape=jax.ShapeDtypeStruct((EP, D_OUT), jnp.float32),
    )(ef, xs, W1, b1.reshape(1, H1), W2, b2.reshape(1, H2), W3q, b3q, rmat)


def _final_body(sum_ref, cnt_ref, x_ref, wr_ref, br_ref, g_ref, bta_ref,
                out_ref):
    ssum = sum_ref[0] + sum_ref[1]
    cnt = cnt_ref[0, :, 0:1] + cnt_ref[1, :, 0:1]
    aggr = ssum / jnp.maximum(cnt, 1.0)
    out = aggr + jnp.dot(x_ref[...], wr_ref[...],
                         preferred_element_type=jnp.float32) + br_ref[...]
    out = jnp.where(out >= 0, out, 0.01 * out)
    mean = jnp.mean(out, axis=0, keepdims=True)
    cen = out - mean
    var = jnp.mean(cen * cen, axis=0, keepdims=True)
    out_ref[...] = g_ref[...] * cen * lax.rsqrt(var + 1e-5) + bta_ref[...]


def _finalize(sum_parts, cnt_parts, x, W_root, b_root, bn_gamma, bn_beta):
    full = lambda i: (0, 0)
    return pl.pallas_call(
        _final_body,
        grid=(1,),
        in_specs=[
            pl.BlockSpec((NC, N, D_OUT), lambda i: (0, 0, 0)),
            pl.BlockSpec((NC, N, 16), lambda i: (0, 0, 0)),
            pl.BlockSpec((N, D_IN), full),
            pl.BlockSpec((D_IN, D_OUT), full),
            pl.BlockSpec((1, D_OUT), full),
            pl.BlockSpec((1, D_OUT), full),
            pl.BlockSpec((1, D_OUT), full),
        ],
        out_specs=pl.BlockSpec((N, D_OUT), full),
        out_shape=jax.ShapeDtypeStruct((N, D_OUT), jnp.float32),
    )(sum_parts, cnt_parts, x, W_root, b_root.reshape(1, D_OUT),
      bn_gamma.reshape(1, D_OUT), bn_beta.reshape(1, D_OUT))


def kernel(x, edge_index, edge_features, W1, b1, W2, b2, W3, b3,
           W_root, b_root, bn_gamma, bn_beta):
    src = edge_index[0]
    dst = edge_index[1]
    src_p = jnp.concatenate([src, jnp.zeros((EP - E,), jnp.int32)])
    dst_p = jnp.concatenate([dst, jnp.full((EP - E,), N, jnp.int32)])
    dst3 = dst_p.reshape(NW, SPJ, SCB)

    ones_arr = jnp.zeros((SCB, 16), jnp.float32).at[:, 0].set(1.0)
    zeros16 = jnp.zeros((NPS, 16), jnp.float32)
    zeros32 = jnp.zeros((NPS, D_OUT), jnp.float32)

    xs, cnt_parts = _gather_count(x, src_p, dst3, ones_arr, zeros16)
    msg = _messages(edge_features, xs, W1, b1, W2, b2, W3, b3)
    sum_parts = _scatter_sum(msg, dst3, zeros32)
    out = _finalize(sum_parts, cnt_parts, x, W_root, b_root, bn_gamma, bn_beta)
    return (out, edge_index, edge_features)
